# baseline (speedup 1.0000x reference)
import jax
import jax.numpy as jnp
from jax import lax
from jax.experimental import pallas as pl
from jax.experimental.pallas import tpu as pltpu
from jax.experimental.pallas import tpu_sc as plsc

N = 10000
E = 320000
IN_DIM = 128
HIDDEN = 64
NUM_GRAPHS = 64

NC = 2
NS = 16
NW = NC * NS
CH = 80
CPW = 125
NPAD = 10240
RPS = NPAD // NS
RB = 2000
NBLK = N // RB


def _sc_mesh():
    return plsc.VectorSubcoreMesh(core_axis_name="c", subcore_axis_name="s")


def _deg_body(dst_hbm, out_hbm, idx_v, ones_v, zrow_v, acc_sh, sem):
    c = lax.axis_index("c")
    s = lax.axis_index("s")
    w = c * NS + s

    @pl.loop(0, CH)
    def _fill(i):
        ones_v[i] = jnp.ones((16,), jnp.float32)
        zrow_v[i] = jnp.zeros((16,), jnp.float32)

    for t in range(RPS // CH):
        pltpu.sync_copy(zrow_v, acc_sh.at[pl.ds(s * RPS + t * CH, CH)])
    pltpu.sync_copy(dst_hbm.at[w], idx_v)
    plsc.subcore_barrier()

    @pl.loop(0, CPW)
    def _scat(k):
        pltpu.sync_copy(ones_v, acc_sh.at[idx_v.at[k]], add=True)

    plsc.subcore_barrier()
    pltpu.sync_copy(acc_sh.at[pl.ds(s * RPS, RPS)],
                    out_hbm.at[c, pl.ds(s * RPS, RPS)])


def _deg_partials(dst3d):
    kern = pl.kernel(
        _deg_body,
        out_type=jax.ShapeDtypeStruct((NC, NPAD, 16), jnp.float32),
        mesh=_sc_mesh(),
        scratch_types=[
            pltpu.VMEM((CPW, CH), jnp.int32),
            pltpu.VMEM((CH, 16), jnp.float32),
            pltpu.VMEM((CH, 16), jnp.float32),
            pltpu.VMEM_SHARED((NPAD, 16), jnp.float32),
            pltpu.SemaphoreType.DMA,
        ],
        compiler_params=pltpu.CompilerParams(use_tc_tiling_on_sc=False),
    )
    return kern(dst3d)


NBUF = 5


def _scatter_body(table_hbm, src_hbm, dst_hbm, out_hbm,
                  srcv, dstv, b0, b1, b2, b3, b4, acc_sh,
                  ga, gb, gc, gd, ge, sa, sb, sc, sd, se):
    bufs = (b0, b1, b2, b3, b4)
    gsem = (ga, gb, gc, gd, ge)
    ssem = (sa, sb, sc, sd, se)
    c = lax.axis_index("c")
    s = lax.axis_index("s")
    w = c * NS + s

    @pl.loop(0, CH)
    def _fill(i):
        for j in range(HIDDEN // 16):
            b0[i, pl.ds(j * 16, 16)] = jnp.zeros((16,), jnp.float32)

    for t in range(RPS // CH):
        pltpu.sync_copy(b0, acc_sh.at[pl.ds(s * RPS + t * CH, CH)])
    pltpu.sync_copy(src_hbm.at[w], srcv)
    pltpu.sync_copy(dst_hbm.at[w], dstv)
    plsc.subcore_barrier()

    for k in range(3):
        pltpu.async_copy(table_hbm.at[srcv.at[k]], bufs[k], gsem[k])

    @pl.loop(0, CPW // NBUF)
    def _outer(ko):
        for b in range(NBUF):
            k = ko * NBUF + b
            nb = (b + 3) % NBUF

            @pl.when(k >= 2)
            def _():
                pltpu.make_async_copy(bufs[nb], acc_sh.at[dstv.at[k - 2]],
                                      ssem[nb]).wait()

            @pl.when(k + 3 < CPW)
            def _():
                pltpu.async_copy(table_hbm.at[srcv.at[k + 3]], bufs[nb],
                                 gsem[nb])

            pltpu.make_async_copy(table_hbm.at[srcv.at[k]], bufs[b],
                                  gsem[b]).wait()
            pltpu.async_copy(bufs[b], acc_sh.at[dstv.at[k]], ssem[b],
                             add=True)

    for k in (CPW - 2, CPW - 1):
        pltpu.make_async_copy(bufs[k % NBUF], acc_sh.at[dstv.at[k]],
                              ssem[k % NBUF]).wait()

    plsc.subcore_barrier()
    pltpu.sync_copy(acc_sh.at[pl.ds(s * RPS, RPS)],
                    out_hbm.at[c, pl.ds(s * RPS, RPS)])


def _edge_scatter(table, src3d, dst3d):
    kern = pl.kernel(
        _scatter_body,
        out_type=jax.ShapeDtypeStruct((NC, NPAD, HIDDEN), jnp.float32),
        mesh=_sc_mesh(),
        scratch_types=(
            [pltpu.VMEM((CPW, CH), jnp.int32)] * 2
            + [pltpu.VMEM((CH, HIDDEN), jnp.float32)] * NBUF
            + [pltpu.VMEM_SHARED((NPAD, HIDDEN), jnp.float32)]
            + [pltpu.SemaphoreType.DMA] * (2 * NBUF)
        ),
        compiler_params=pltpu.CompilerParams(use_tc_tiling_on_sc=False),
    )
    return kern(table, src3d, dst3d)


def _mm1_body(x_ref, w1_ref, h_ref):
    h_ref[...] = jnp.dot(x_ref[...], w1_ref[...],
                         preferred_element_type=jnp.float32)


def _tc_mm1(x, W1):
    return pl.pallas_call(
        _mm1_body,
        grid=(NBLK,),
        in_specs=[
            pl.BlockSpec((RB, IN_DIM), lambda i: (i, 0)),
            pl.BlockSpec((IN_DIM, HIDDEN), lambda i: (0, 0)),
        ],
        out_specs=pl.BlockSpec((RB, HIDDEN), lambda i: (i, 0)),
        out_shape=jax.ShapeDtypeStruct((N, HIDDEN), jnp.float32),
    )(x, W1)


def _scale_body(dp_ref, h_ref, g1_ref, dis_ref):
    dp = dp_ref[...]
    deg = dp[0, :, 0:1] + dp[1, :, 0:1] + 1.0
    dis = 1.0 / jnp.sqrt(deg)
    g1_ref[...] = h_ref[...] * dis
    dis_ref[...] = dis


def _tc_scale(dp, h1raw):
    return pl.pallas_call(
        _scale_body,
        grid=(NBLK,),
        in_specs=[
            pl.BlockSpec((NC, RB, 16), lambda i: (0, i, 0)),
            pl.BlockSpec((RB, HIDDEN), lambda i: (i, 0)),
        ],
        out_specs=[
            pl.BlockSpec((RB, HIDDEN), lambda i: (i, 0)),
            pl.BlockSpec((RB, 1), lambda i: (i, 0)),
        ],
        out_shape=[
            jax.ShapeDtypeStruct((N, HIDDEN), jnp.float32),
            jax.ShapeDtypeStruct((N, 1), jnp.float32),
        ],
    )(dp, h1raw)


def _mid_body(p_ref, g1_ref, dis_ref, b1_ref, w2_ref, g2_ref):
    p = p_ref[...]
    dis = dis_ref[...]
    h1 = jnp.maximum((p[0] + p[1] + g1_ref[...]) * dis + b1_ref[...], 0.0)
    g2_ref[...] = jnp.dot(h1, w2_ref[...],
                          preferred_element_type=jnp.float32) * dis


def _tc_mid(P, g1, dis, b1r, W2):
    return pl.pallas_call(
        _mid_body,
        grid=(NBLK,),
        in_specs=[
            pl.BlockSpec((NC, RB, HIDDEN), lambda i: (0, i, 0)),
            pl.BlockSpec((RB, HIDDEN), lambda i: (i, 0)),
            pl.BlockSpec((RB, 1), lambda i: (i, 0)),
            pl.BlockSpec((1, HIDDEN), lambda i: (0, 0)),
            pl.BlockSpec((HIDDEN, HIDDEN), lambda i: (0, 0)),
        ],
        out_specs=pl.BlockSpec((RB, HIDDEN), lambda i: (i, 0)),
        out_shape=jax.ShapeDtypeStruct((N, HIDDEN), jnp.float32),
    )(P, g1, dis, b1r, W2)


def _pool_body(q_ref, g2_ref, dis_ref, b2_ref, batch_ref, out_ref, acc, cnt):
    i = pl.program_id(0)

    @pl.when(i == 0)
    def _():
        acc[...] = jnp.zeros_like(acc)
        cnt[...] = jnp.zeros_like(cnt)

    q = q_ref[...]
    h2 = jnp.maximum((q[0] + q[1] + g2_ref[...]) * dis_ref[...] + b2_ref[...],
                     0.0)
    onehot = (batch_ref[...] ==
              lax.broadcasted_iota(jnp.int32, (1, NUM_GRAPHS), 1)
              ).astype(jnp.float32)
    dn = (((0,), (0,)), ((), ()))
    acc[...] += lax.dot_general(onehot, h2, dn,
                                preferred_element_type=jnp.float32)
    cnt[...] += lax.dot_general(onehot, jnp.ones((RB, NUM_GRAPHS),
                                                 jnp.float32), dn,
                                preferred_element_type=jnp.float32)

    @pl.when(i == NBLK - 1)
    def _():
        out_ref[...] = acc[...] / jnp.maximum(cnt[...], 1.0)


def _tc_pool(Q, g2, dis, b2r, batch2d):
    return pl.pallas_call(
        _pool_body,
        grid=(NBLK,),
        in_specs=[
            pl.BlockSpec((NC, RB, HIDDEN), lambda i: (0, i, 0)),
            pl.BlockSpec((RB, HIDDEN), lambda i: (i, 0)),
            pl.BlockSpec((RB, 1), lambda i: (i, 0)),
            pl.BlockSpec((1, HIDDEN), lambda i: (0, 0)),
            pl.BlockSpec((RB, 1), lambda i: (i, 0)),
        ],
        out_specs=pl.BlockSpec((NUM_GRAPHS, HIDDEN), lambda i: (0, 0)),
        out_shape=jax.ShapeDtypeStruct((NUM_GRAPHS, HIDDEN), jnp.float32),
        scratch_shapes=[
            pltpu.VMEM((NUM_GRAPHS, HIDDEN), jnp.float32),
            pltpu.VMEM((NUM_GRAPHS, NUM_GRAPHS), jnp.float32),
        ],
    )(Q, g2, dis, b2r, batch2d)


@jax.jit
def kernel(x, edge_index, batch, W1, b1, W2, b2):
    src3d = edge_index[0].reshape(NW, CPW, CH)
    dst3d = edge_index[1].reshape(NW, CPW, CH)
    batch2d = batch.reshape(N, 1)
    b1r = b1.reshape(1, HIDDEN)
    b2r = b2.reshape(1, HIDDEN)

    dp = _deg_partials(dst3d)
    h1raw = _tc_mm1(x, W1)
    g1, dis = _tc_scale(dp, h1raw)
    P = _edge_scatter(g1, src3d, dst3d)
    g2 = _tc_mid(P, g1, dis, b1r, W2)
    Q = _edge_scatter(g2, src3d, dst3d)
    return _tc_pool(Q, g2, dis, b2r, batch2d)


# speedup vs baseline: 1.0261x; 1.0261x over previous
import jax
import jax.numpy as jnp
from jax import lax
from jax.experimental import pallas as pl
from jax.experimental.pallas import tpu as pltpu
from jax.experimental.pallas import tpu_sc as plsc

N = 10000
E = 320000
IN_DIM = 128
HIDDEN = 64
NUM_GRAPHS = 64

NC = 2
NS = 16
NW = NC * NS
CH = 80
CPW = 125
NPAD = 10240
RPS = NPAD // NS
RB = 2000
NBLK = N // RB


def _sc_mesh():
    return plsc.VectorSubcoreMesh(core_axis_name="c", subcore_axis_name="s")


def _deg_body(dst_hbm, out_hbm, idx_v, ones_v, zrow_v, acc_sh, sem, sem2):
    c = lax.axis_index("c")
    s = lax.axis_index("s")
    w = c * NS + s

    @pl.loop(0, CH)
    def _fill(i):
        ones_v[i] = jnp.ones((16,), jnp.float32)
        zrow_v[i] = jnp.zeros((16,), jnp.float32)

    for t in range(RPS // CH):
        pltpu.sync_copy(zrow_v, acc_sh.at[pl.ds(s * RPS + t * CH, CH)])
    pltpu.sync_copy(dst_hbm.at[w], idx_v)
    plsc.subcore_barrier()

    pltpu.async_copy(ones_v, acc_sh.at[idx_v.at[0]], sem, add=True)
    pltpu.async_copy(ones_v, acc_sh.at[idx_v.at[1]], sem2, add=True)

    @pl.loop(0, CPW - 2)
    def _scat(k):
        even = k % 2 == 0

        @pl.when(even)
        def _():
            pltpu.make_async_copy(ones_v, acc_sh.at[idx_v.at[k]], sem).wait()
            pltpu.async_copy(ones_v, acc_sh.at[idx_v.at[k + 2]], sem,
                             add=True)

        @pl.when(jnp.logical_not(even))
        def _():
            pltpu.make_async_copy(ones_v, acc_sh.at[idx_v.at[k]], sem2).wait()
            pltpu.async_copy(ones_v, acc_sh.at[idx_v.at[k + 2]], sem2,
                             add=True)

    lp = (CPW - 2) % 2
    if lp == 0:
        pltpu.make_async_copy(ones_v, acc_sh.at[idx_v.at[CPW - 2]],
                              sem).wait()
        pltpu.make_async_copy(ones_v, acc_sh.at[idx_v.at[CPW - 1]],
                              sem2).wait()
    else:
        pltpu.make_async_copy(ones_v, acc_sh.at[idx_v.at[CPW - 2]],
                              sem2).wait()
        pltpu.make_async_copy(ones_v, acc_sh.at[idx_v.at[CPW - 1]],
                              sem).wait()

    plsc.subcore_barrier()
    pltpu.sync_copy(acc_sh.at[pl.ds(s * RPS, RPS)],
                    out_hbm.at[c, pl.ds(s * RPS, RPS)])


def _deg_partials(dst3d):
    kern = pl.kernel(
        _deg_body,
        out_type=jax.ShapeDtypeStruct((NC, NPAD, 16), jnp.float32),
        mesh=_sc_mesh(),
        scratch_types=[
            pltpu.VMEM((CPW, CH), jnp.int32),
            pltpu.VMEM((CH, 16), jnp.float32),
            pltpu.VMEM((CH, 16), jnp.float32),
            pltpu.VMEM_SHARED((NPAD, 16), jnp.float32),
            pltpu.SemaphoreType.DMA,
            pltpu.SemaphoreType.DMA,
        ],
        compiler_params=pltpu.CompilerParams(use_tc_tiling_on_sc=False),
    )
    return kern(dst3d)


NBUF = 5


def _scatter_body(table_hbm, src_hbm, dst_hbm, out_hbm,
                  srcv, dstv, zbuf, b0, b1, b2, b3, b4, acc_sh,
                  ga, gb, gc, gd, ge, sa, sb, sc, sd, se):
    bufs = (b0, b1, b2, b3, b4)
    gsem = (ga, gb, gc, gd, ge)
    ssem = (sa, sb, sc, sd, se)
    c = lax.axis_index("c")
    s = lax.axis_index("s")
    w = c * NS + s

    pltpu.sync_copy(src_hbm.at[w], srcv)
    pltpu.sync_copy(dst_hbm.at[w], dstv)
    for k in range(3):
        pltpu.async_copy(table_hbm.at[srcv.at[k]], bufs[k], gsem[k])

    @pl.loop(0, CH)
    def _fill(i):
        for j in range(HIDDEN // 16):
            zbuf[i, pl.ds(j * 16, 16)] = jnp.zeros((16,), jnp.float32)

    for t in range(RPS // CH):
        pltpu.sync_copy(zbuf, acc_sh.at[pl.ds(s * RPS + t * CH, CH)])
    plsc.subcore_barrier()

    @pl.loop(0, CPW // NBUF)
    def _outer(ko):
        for b in range(NBUF):
            k = ko * NBUF + b
            nb = (b + 3) % NBUF

            @pl.when(k >= 2)
            def _():
                pltpu.make_async_copy(bufs[nb], acc_sh.at[dstv.at[k - 2]],
                                      ssem[nb]).wait()

            @pl.when(k + 3 < CPW)
            def _():
                pltpu.async_copy(table_hbm.at[srcv.at[k + 3]], bufs[nb],
                                 gsem[nb])

            pltpu.make_async_copy(table_hbm.at[srcv.at[k]], bufs[b],
                                  gsem[b]).wait()
            pltpu.async_copy(bufs[b], acc_sh.at[dstv.at[k]], ssem[b],
                             add=True)

    for k in (CPW - 2, CPW - 1):
        pltpu.make_async_copy(bufs[k % NBUF], acc_sh.at[dstv.at[k]],
                              ssem[k % NBUF]).wait()

    plsc.subcore_barrier()
    pltpu.sync_copy(acc_sh.at[pl.ds(s * RPS, RPS)],
                    out_hbm.at[c, pl.ds(s * RPS, RPS)])


def _edge_scatter(table, src3d, dst3d):
    kern = pl.kernel(
        _scatter_body,
        out_type=jax.ShapeDtypeStruct((NC, NPAD, HIDDEN), jnp.float32),
        mesh=_sc_mesh(),
        scratch_types=(
            [pltpu.VMEM((CPW, CH), jnp.int32)] * 2
            + [pltpu.VMEM((CH, HIDDEN), jnp.float32)] * (NBUF + 1)
            + [pltpu.VMEM_SHARED((NPAD, HIDDEN), jnp.float32)]
            + [pltpu.SemaphoreType.DMA] * (2 * NBUF)
        ),
        compiler_params=pltpu.CompilerParams(use_tc_tiling_on_sc=False),
    )
    return kern(table, src3d, dst3d)


def _mm1_body(x_ref, w1_ref, batch_ref, h_ref, cinv_ref, cacc):
    i = pl.program_id(0)

    @pl.when(i == 0)
    def _():
        cacc[...] = jnp.zeros_like(cacc)

    h_ref[...] = jnp.dot(x_ref[...], w1_ref[...],
                         preferred_element_type=jnp.float32)
    onehot = (batch_ref[...] ==
              lax.broadcasted_iota(jnp.int32, (1, NUM_GRAPHS), 1)
              ).astype(jnp.float32)
    dn = (((0,), (0,)), ((), ()))
    cacc[...] += lax.dot_general(onehot, jnp.ones((RB, HIDDEN), jnp.float32),
                                 dn, preferred_element_type=jnp.float32)

    @pl.when(i == NBLK - 1)
    def _():
        cinv_ref[...] = 1.0 / jnp.maximum(cacc[...], 1.0)


def _tc_mm1(x, W1, batch2d):
    return pl.pallas_call(
        _mm1_body,
        grid=(NBLK,),
        in_specs=[
            pl.BlockSpec((RB, IN_DIM), lambda i: (i, 0)),
            pl.BlockSpec((IN_DIM, HIDDEN), lambda i: (0, 0)),
            pl.BlockSpec((RB, 1), lambda i: (i, 0)),
        ],
        out_specs=[
            pl.BlockSpec((RB, HIDDEN), lambda i: (i, 0)),
            pl.BlockSpec((NUM_GRAPHS, HIDDEN), lambda i: (0, 0)),
        ],
        out_shape=[
            jax.ShapeDtypeStruct((N, HIDDEN), jnp.float32),
            jax.ShapeDtypeStruct((NUM_GRAPHS, HIDDEN), jnp.float32),
        ],
        scratch_shapes=[pltpu.VMEM((NUM_GRAPHS, HIDDEN), jnp.float32)],
    )(x, W1, batch2d)


def _scale_body(dp_ref, h_ref, g1_ref, dis_ref):
    dp = dp_ref[...]
    deg = dp[0, :, 0:1] + dp[1, :, 0:1] + 1.0
    dis = 1.0 / jnp.sqrt(deg)
    g1_ref[...] = h_ref[...] * dis
    dis_ref[...] = dis


def _tc_scale(dp, h1raw):
    return pl.pallas_call(
        _scale_body,
        grid=(NBLK,),
        in_specs=[
            pl.BlockSpec((NC, RB, 16), lambda i: (0, i, 0)),
            pl.BlockSpec((RB, HIDDEN), lambda i: (i, 0)),
        ],
        out_specs=[
            pl.BlockSpec((RB, HIDDEN), lambda i: (i, 0)),
            pl.BlockSpec((RB, 1), lambda i: (i, 0)),
        ],
        out_shape=[
            jax.ShapeDtypeStruct((N, HIDDEN), jnp.float32),
            jax.ShapeDtypeStruct((N, 1), jnp.float32),
        ],
    )(dp, h1raw)


def _mid_body(p_ref, g1_ref, dis_ref, b1_ref, w2_ref, g2_ref):
    p = p_ref[...]
    dis = dis_ref[...]
    h1 = jnp.maximum((p[0] + p[1] + g1_ref[...]) * dis + b1_ref[...], 0.0)
    g2_ref[...] = jnp.dot(h1, w2_ref[...],
                          preferred_element_type=jnp.float32) * dis


def _tc_mid(P, g1, dis, b1r, W2):
    return pl.pallas_call(
        _mid_body,
        grid=(NBLK,),
        in_specs=[
            pl.BlockSpec((NC, RB, HIDDEN), lambda i: (0, i, 0)),
            pl.BlockSpec((RB, HIDDEN), lambda i: (i, 0)),
            pl.BlockSpec((RB, 1), lambda i: (i, 0)),
            pl.BlockSpec((1, HIDDEN), lambda i: (0, 0)),
            pl.BlockSpec((HIDDEN, HIDDEN), lambda i: (0, 0)),
        ],
        out_specs=pl.BlockSpec((RB, HIDDEN), lambda i: (i, 0)),
        out_shape=jax.ShapeDtypeStruct((N, HIDDEN), jnp.float32),
    )(P, g1, dis, b1r, W2)


def _pool_body(q_ref, g2_ref, dis_ref, b2_ref, batch_ref, cinv_ref, out_ref,
               acc):
    i = pl.program_id(0)

    @pl.when(i == 0)
    def _():
        acc[...] = jnp.zeros_like(acc)

    q = q_ref[...]
    h2 = jnp.maximum((q[0] + q[1] + g2_ref[...]) * dis_ref[...] + b2_ref[...],
                     0.0)
    onehot = (batch_ref[...] ==
              lax.broadcasted_iota(jnp.int32, (1, NUM_GRAPHS), 1)
              ).astype(jnp.float32)
    dn = (((0,), (0,)), ((), ()))
    acc[...] += lax.dot_general(onehot, h2, dn,
                                preferred_element_type=jnp.float32)

    @pl.when(i == NBLK - 1)
    def _():
        out_ref[...] = acc[...] * cinv_ref[...]


def _tc_pool(Q, g2, dis, b2r, batch2d, cinv):
    return pl.pallas_call(
        _pool_body,
        grid=(NBLK,),
        in_specs=[
            pl.BlockSpec((NC, RB, HIDDEN), lambda i: (0, i, 0)),
            pl.BlockSpec((RB, HIDDEN), lambda i: (i, 0)),
            pl.BlockSpec((RB, 1), lambda i: (i, 0)),
            pl.BlockSpec((1, HIDDEN), lambda i: (0, 0)),
            pl.BlockSpec((RB, 1), lambda i: (i, 0)),
            pl.BlockSpec((NUM_GRAPHS, HIDDEN), lambda i: (0, 0)),
        ],
        out_specs=pl.BlockSpec((NUM_GRAPHS, HIDDEN), lambda i: (0, 0)),
        out_shape=jax.ShapeDtypeStruct((NUM_GRAPHS, HIDDEN), jnp.float32),
        scratch_shapes=[
            pltpu.VMEM((NUM_GRAPHS, HIDDEN), jnp.float32),
        ],
    )(Q, g2, dis, b2r, batch2d, cinv)


@jax.jit
def kernel(x, edge_index, batch, W1, b1, W2, b2):
    src3d = edge_index[0].reshape(NW, CPW, CH)
    dst3d = edge_index[1].reshape(NW, CPW, CH)
    batch2d = batch.reshape(N, 1)
    b1r = b1.reshape(1, HIDDEN)
    b2r = b2.reshape(1, HIDDEN)

    dp = _deg_partials(dst3d)
    h1raw, cinv = _tc_mm1(x, W1, batch2d)
    g1, dis = _tc_scale(dp, h1raw)
    P = _edge_scatter(g1, src3d, dst3d)
    g2 = _tc_mid(P, g1, dis, b1r, W2)
    Q = _edge_scatter(g2, src3d, dst3d)
    return _tc_pool(Q, g2, dis, b2r, batch2d, cinv)
